# baseline (speedup 1.0000x reference)
import functools

import jax
import jax.numpy as jnp
from jax import lax
from jax.experimental import pallas as pl
from jax.experimental.pallas import tpu as pltpu
from jax.experimental.pallas import tpu_sc as plsc


def _sc_gather(table, idx):
    V, D = table.shape
    B = idx.shape[0]
    info = plsc.get_sparse_core_info()
    num_workers = info.num_cores * info.num_subcores
    b_per_w = B // num_workers
    mesh = plsc.VectorSubcoreMesh(core_axis_name="c", subcore_axis_name="s")

    @functools.partial(
        pl.kernel,
        mesh=mesh,
        out_type=jax.ShapeDtypeStruct((B, D), jnp.float32),
        scratch_types=[
            pltpu.VMEM((b_per_w,), jnp.int32),
            pltpu.VMEM((b_per_w, D), jnp.float32),
            pltpu.SemaphoreType.DMA,
        ],
    )
    def gather_kernel(table_hbm, idx_hbm, out_hbm, idx_v, rows_v, sem):
        wid = lax.axis_index("s") * info.num_cores + lax.axis_index("c")
        base = wid * b_per_w
        pltpu.sync_copy(idx_hbm.at[pl.ds(base, b_per_w)], idx_v)
        pltpu.async_copy(table_hbm.at[idx_v], rows_v, sem).wait()
        pltpu.sync_copy(rows_v, out_hbm.at[pl.ds(base, b_per_w)])

    return gather_kernel(table, idx)


def _select_body(wp_ref, par_ref, o_ref):
    half = o_ref.shape[1]
    par = par_ref[...]
    o_ref[...] = jnp.where(par != 0, wp_ref[:, half:], wp_ref[:, :half])


def _broadcast_body(lat_ref, wg_ref, wov_ref, o1_ref, o2_ref):
    lat = lat_ref[...]
    o1_ref[...] = lat * wg_ref[...]
    o2_ref[...] = lat * wov_ref[...]


def kernel(latent, genes_oi, W_height, W_overall):
    C = latent.shape[0]
    G = genes_oi.shape[0]
    V, D = W_height.shape
    N = W_overall.shape[0]

    gi = genes_oi.astype(jnp.int32)
    packed = W_height.reshape(V // 2, 2 * D)
    wp = _sc_gather(packed, gi >> 1)

    sel = pl.pallas_call(
        _select_body,
        out_shape=jax.ShapeDtypeStruct((G, D), jnp.float32),
    )(wp, (gi & 1).reshape(G, 1))

    lat2 = latent.reshape(C, 1)
    wgf = sel.reshape(1, G * D)
    wovf = W_overall.reshape(1, N)

    CB = 8
    out1, out2 = pl.pallas_call(
        _broadcast_body,
        grid=(C // CB,),
        in_specs=[
            pl.BlockSpec((CB, 1), lambda i: (i, 0)),
            pl.BlockSpec((1, G * D), lambda i: (0, 0)),
            pl.BlockSpec((1, N), lambda i: (0, 0)),
        ],
        out_specs=[
            pl.BlockSpec((CB, G * D), lambda i: (i, 0)),
            pl.BlockSpec((CB, N), lambda i: (i, 0)),
        ],
        out_shape=[
            jax.ShapeDtypeStruct((C, G * D), jnp.float32),
            jax.ShapeDtypeStruct((C, N), jnp.float32),
        ],
    )(lat2, wgf, wovf)

    return out1.reshape(C, G, D), out2.reshape(C, N, 1)


# speedup vs baseline: 1.3385x; 1.3385x over previous
import functools

import jax
import jax.numpy as jnp
from jax import lax
from jax.experimental import pallas as pl
from jax.experimental.pallas import tpu as pltpu
from jax.experimental.pallas import tpu_sc as plsc


def _sc_gather(table, idx):
    V, D = table.shape
    B = idx.shape[0]
    info = plsc.get_sparse_core_info()
    num_workers = info.num_cores * info.num_subcores
    b_per_w = B // num_workers
    mesh = plsc.VectorSubcoreMesh(core_axis_name="c", subcore_axis_name="s")

    @functools.partial(
        pl.kernel,
        mesh=mesh,
        out_type=jax.ShapeDtypeStruct((B, D), jnp.float32),
        scratch_types=[
            pltpu.VMEM((b_per_w,), jnp.int32),
            pltpu.VMEM((b_per_w, D), jnp.float32),
            pltpu.SemaphoreType.DMA,
        ],
    )
    def gather_kernel(table_hbm, idx_hbm, out_hbm, idx_v, rows_v, sem):
        wid = lax.axis_index("s") * info.num_cores + lax.axis_index("c")
        base = wid * b_per_w
        pltpu.sync_copy(idx_hbm.at[pl.ds(base, b_per_w)], idx_v)
        pltpu.async_copy(table_hbm.at[idx_v], rows_v, sem).wait()
        pltpu.sync_copy(rows_v, out_hbm.at[pl.ds(base, b_per_w)])

    return gather_kernel(table, idx)


def _select_t_body(wp_ref, par_ref, o_ref):
    half = o_ref.shape[0]
    par = par_ref[...]
    sel = jnp.where(par != 0, wp_ref[:, half:], wp_ref[:, :half])
    o_ref[...] = sel.T


def _overall_body(wov2_ref, lat2_ref, o_ref):
    rows = o_ref.shape[0]
    parity = lax.broadcasted_iota(jnp.int32, (rows, 1), 0) & 1
    lat0 = lat2_ref[0:1, :]
    lat1 = lat2_ref[1:2, :]
    o_ref[...] = wov2_ref[...] * jnp.where(parity == 0, lat0, lat1)


def _height_body(lat_ref, wgt_ref, o_ref):
    o_ref[...] = lat_ref[...] * wgt_ref[...]


def kernel(latent, genes_oi, W_height, W_overall):
    C = latent.shape[0]
    G = genes_oi.shape[0]
    V, D = W_height.shape
    N = W_overall.shape[0]

    gi = genes_oi.astype(jnp.int32)
    packed = W_height.reshape(V // 2, 2 * D)
    wp = _sc_gather(packed, gi >> 1)

    wgt = pl.pallas_call(
        _select_t_body,
        out_shape=jax.ShapeDtypeStruct((D, G), jnp.float32),
    )(wp, (gi & 1).reshape(G, 1))

    LANES = 128
    J = C // LANES
    wov2 = jnp.repeat(W_overall.reshape(N, 1), J, axis=0)
    lat2 = latent.reshape(J, LANES)
    NB = 10000
    out2 = pl.pallas_call(
        _overall_body,
        grid=(J * N // NB,),
        in_specs=[
            pl.BlockSpec((NB, 1), lambda i: (i, 0)),
            pl.BlockSpec((J, LANES), lambda i: (0, 0)),
        ],
        out_specs=pl.BlockSpec((NB, LANES), lambda i: (i, 0)),
        out_shape=jax.ShapeDtypeStruct((J * N, LANES), jnp.float32),
    )(wov2, lat2)

    CB = 8
    out1 = pl.pallas_call(
        _height_body,
        grid=(C // CB,),
        in_specs=[
            pl.BlockSpec((CB, 1, 1), lambda i: (i, 0, 0)),
            pl.BlockSpec((1, D, G), lambda i: (0, 0, 0)),
        ],
        out_specs=pl.BlockSpec((CB, D, G), lambda i: (i, 0, 0)),
        out_shape=jax.ShapeDtypeStruct((C, D, G), jnp.float32),
    )(latent.reshape(C, 1, 1), wgt.reshape(1, D, G))

    delta_height = out1.transpose(0, 2, 1)
    delta_overall = out2.reshape(N, 1, C).transpose(2, 0, 1)
    return delta_height, delta_overall


# speedup vs baseline: 1.8080x; 1.3508x over previous
import functools

import jax
import jax.numpy as jnp
from jax import lax
from jax.experimental import pallas as pl
from jax.experimental.pallas import tpu as pltpu
from jax.experimental.pallas import tpu_sc as plsc


def _sc_gather(table, idx):
    V, D = table.shape
    B = idx.shape[0]
    info = plsc.get_sparse_core_info()
    num_workers = info.num_cores * info.num_subcores
    b_per_w = B // num_workers
    mesh = plsc.VectorSubcoreMesh(core_axis_name="c", subcore_axis_name="s")

    @functools.partial(
        pl.kernel,
        mesh=mesh,
        out_type=jax.ShapeDtypeStruct((B, D), jnp.float32),
        scratch_types=[
            pltpu.VMEM((b_per_w,), jnp.int32),
            pltpu.VMEM((b_per_w, D), jnp.float32),
            pltpu.SemaphoreType.DMA,
        ],
    )
    def gather_kernel(table_hbm, idx_hbm, out_hbm, idx_v, rows_v, sem):
        wid = lax.axis_index("s") * info.num_cores + lax.axis_index("c")
        base = wid * b_per_w
        pltpu.sync_copy(idx_hbm.at[pl.ds(base, b_per_w)], idx_v)
        pltpu.async_copy(table_hbm.at[idx_v], rows_v, sem).wait()
        pltpu.sync_copy(rows_v, out_hbm.at[pl.ds(base, b_per_w)])

    return gather_kernel(table, idx)


def _select_t_body(wp_ref, par_ref, o_ref):
    half = o_ref.shape[0]
    par = par_ref[...]
    sel = jnp.where(par != 0, wp_ref[:, half:], wp_ref[:, :half])
    o_ref[...] = sel.T


def _overall_body(lat_ref, wov_ref, o_ref):
    o_ref[...] = lat_ref[...] * wov_ref[...]


def _height_body(lat_ref, wgt_ref, o_ref):
    o_ref[...] = lat_ref[...] * wgt_ref[...]


def kernel(latent, genes_oi, W_height, W_overall):
    C = latent.shape[0]
    G = genes_oi.shape[0]
    V, D = W_height.shape
    N = W_overall.shape[0]

    gi = genes_oi.astype(jnp.int32)
    packed = W_height.reshape(V // 2, 2 * D)
    wp = _sc_gather(packed, gi >> 1)

    wgt = pl.pallas_call(
        _select_t_body,
        out_shape=jax.ShapeDtypeStruct((D, G), jnp.float32),
    )(wp, (gi & 1).reshape(G, 1))

    CB2 = 8
    out2 = pl.pallas_call(
        _overall_body,
        grid=(C // CB2,),
        in_specs=[
            pl.BlockSpec((CB2, 1), lambda i: (i, 0)),
            pl.BlockSpec((1, N), lambda i: (0, 0)),
        ],
        out_specs=pl.BlockSpec((CB2, N), lambda i: (i, 0)),
        out_shape=jax.ShapeDtypeStruct((C, N), jnp.float32),
    )(latent.reshape(C, 1), W_overall.reshape(1, N))

    CB = 8
    out1 = pl.pallas_call(
        _height_body,
        grid=(C // CB,),
        in_specs=[
            pl.BlockSpec((CB, 1, 1), lambda i: (i, 0, 0)),
            pl.BlockSpec((1, D, G), lambda i: (0, 0, 0)),
        ],
        out_specs=pl.BlockSpec((CB, D, G), lambda i: (i, 0, 0)),
        out_shape=jax.ShapeDtypeStruct((C, D, G), jnp.float32),
    )(latent.reshape(C, 1, 1), wgt.reshape(1, D, G))

    delta_height = out1.transpose(0, 2, 1)
    delta_overall = out2.reshape(C, N, 1)
    return delta_height, delta_overall
